# baseline (speedup 1.0000x reference)
import math

import jax
import jax.numpy as jnp
import numpy as np
from jax import lax
from jax.experimental import pallas as pl
from jax.experimental.pallas import tpu as pltpu
from jax.experimental.pallas import tpu_sc as plsc

VOCAB_SIZE = 1000000
DIM = 64
MAX_SEQ_LEN = 200
BATCH = 1024
SEQ_LEN = 200

_NC = 2
_NS = 16
_NW = _NC * _NS
_BPW = BATCH // _NW
_HALF = SEQ_LEN // 2


def _position_encoding() -> jnp.ndarray:
    i = np.arange(MAX_SEQ_LEN, dtype=np.float64)[:, None]
    j = np.arange(DIM, dtype=np.float64)[None, :]
    even_mask = (np.arange(DIM) % 2 == 0)[None, :]
    temp_even = np.exp(-(j / DIM) * math.log(10000.0))
    temp_odd = np.exp(-((j - 1.0) / DIM) * math.log(10000.0))
    pe = np.where(even_mask, np.sin(i * temp_even), np.cos(i * temp_odd))
    return jnp.asarray(pe[:SEQ_LEN], dtype=jnp.float32)


def _body(x_hbm, pe_hbm, table_hbm, out_hbm, pe_v, idx_v, dest_v, sem):
    c = lax.axis_index("c")
    s = lax.axis_index("s")
    wid = s * _NC + c

    pltpu.sync_copy(pe_hbm, pe_v)

    def per_batch(i, carry):
        b = wid * _BPW + i
        pltpu.sync_copy(x_hbm.at[b], idx_v)
        cp0 = pltpu.async_copy(
            table_hbm.at[idx_v.at[0]], dest_v.at[pl.ds(0, _HALF)], sem)
        cp1 = pltpu.async_copy(
            table_hbm.at[idx_v.at[1]], dest_v.at[pl.ds(_HALF, _HALF)], sem)
        cp0.wait()
        cp1.wait()

        def add_row(r, carry2):
            for d in range(DIM // 16):
                sl = pl.ds(d * 16, 16)
                plsc.addupdate(dest_v.at[r, sl], pe_v[r, sl])
            return carry2

        lax.fori_loop(0, SEQ_LEN, add_row, 0, unroll=2)
        pltpu.sync_copy(dest_v, out_hbm.at[b])
        return carry

    lax.fori_loop(0, _BPW, per_batch, 0)


def kernel(x, table):
    pe = _position_encoding()
    x3 = x.reshape(BATCH, 2, _HALF)
    mesh = plsc.VectorSubcoreMesh(core_axis_name="c", subcore_axis_name="s")
    out = pl.kernel(
        _body,
        out_type=jax.ShapeDtypeStruct((BATCH, SEQ_LEN, DIM), jnp.float32),
        mesh=mesh,
        scratch_types=[
            pltpu.VMEM((SEQ_LEN, DIM), jnp.float32),
            pltpu.VMEM((2, _HALF), jnp.int32),
            pltpu.VMEM((SEQ_LEN, DIM), jnp.float32),
            pltpu.SemaphoreType.DMA,
        ],
        compiler_params=pltpu.CompilerParams(use_tc_tiling_on_sc=False),
    )(x3, pe, table)
    return out


# speedup vs baseline: 1.0645x; 1.0645x over previous
import math

import jax
import jax.numpy as jnp
import numpy as np
from jax import lax
from jax.experimental import pallas as pl
from jax.experimental.pallas import tpu as pltpu
from jax.experimental.pallas import tpu_sc as plsc

VOCAB_SIZE = 1000000
DIM = 64
MAX_SEQ_LEN = 200
BATCH = 1024
SEQ_LEN = 200

_NC = 2
_NS = 16
_NW = _NC * _NS
_BPW = BATCH // _NW
_HALF = SEQ_LEN // 2
_VREGS = SEQ_LEN * DIM // 16


def _position_encoding() -> jnp.ndarray:
    i = np.arange(MAX_SEQ_LEN, dtype=np.float64)[:, None]
    j = np.arange(DIM, dtype=np.float64)[None, :]
    even_mask = (np.arange(DIM) % 2 == 0)[None, :]
    temp_even = np.exp(-(j / DIM) * math.log(10000.0))
    temp_odd = np.exp(-((j - 1.0) / DIM) * math.log(10000.0))
    pe = np.where(even_mask, np.sin(i * temp_even), np.cos(i * temp_odd))
    return jnp.asarray(pe[:SEQ_LEN], dtype=jnp.float32)


def _body(x_hbm, pe_hbm, table_hbm, out_hbm,
          pe_v, idx_v, buf0, buf1, psem, g0, g1, w0, w1):
    c = lax.axis_index("c")
    s = lax.axis_index("s")
    wid = s * _NC + c

    cp_pe = pltpu.async_copy(pe_hbm, pe_v, psem)
    cp_idx = pltpu.async_copy(x_hbm.at[wid], idx_v, psem)
    cp_pe.wait()
    cp_idx.wait()

    bufs = (buf0, buf1)
    gsems = (g0, g1)
    wsems = (w0, w1)

    def fire_gather(i, buf, sem):
        cpa = pltpu.async_copy(
            table_hbm.at[idx_v.at[2 * i]], buf.at[pl.ds(0, _HALF)], sem)
        cpb = pltpu.async_copy(
            table_hbm.at[idx_v.at[2 * i + 1]], buf.at[pl.ds(_HALF, _HALF)],
            sem)
        return cpa, cpb

    def add_pe(buf):
        @plsc.parallel_loop(0, _VREGS, unroll=8)
        def _(j):
            r = j // (DIM // 16)
            col = (j % (DIM // 16)) * 16
            sl = pl.ds(col, 16)
            plsc.addupdate(buf.at[r, sl], pe_v[r, sl])

    pend_g = [None, None]
    pend_w = [None, None]
    pend_g[0] = fire_gather(0, buf0, g0)

    for i in range(_BPW):
        cur = i & 1
        nxt = 1 - cur
        if i + 1 < _BPW:
            if pend_w[nxt] is not None:
                pend_w[nxt].wait()
            pend_g[nxt] = fire_gather(i + 1, bufs[nxt], gsems[nxt])
        cpa, cpb = pend_g[cur]
        cpa.wait()
        cpb.wait()
        add_pe(bufs[cur])
        pend_w[cur] = pltpu.async_copy(
            bufs[cur], out_hbm.at[wid * _BPW + i], wsems[cur])

    pend_w[0].wait()
    pend_w[1].wait()


def kernel(x, table):
    pe = _position_encoding()
    x4 = x.reshape(_NW, 2 * _BPW, _HALF)
    mesh = plsc.VectorSubcoreMesh(core_axis_name="c", subcore_axis_name="s")
    out = pl.kernel(
        _body,
        out_type=jax.ShapeDtypeStruct((BATCH, SEQ_LEN, DIM), jnp.float32),
        mesh=mesh,
        scratch_types=[
            pltpu.VMEM((SEQ_LEN, DIM), jnp.float32),
            pltpu.VMEM((2 * _BPW, _HALF), jnp.int32),
            pltpu.VMEM((SEQ_LEN, DIM), jnp.float32),
            pltpu.VMEM((SEQ_LEN, DIM), jnp.float32),
            pltpu.SemaphoreType.DMA,
            pltpu.SemaphoreType.DMA,
            pltpu.SemaphoreType.DMA,
            pltpu.SemaphoreType.DMA,
            pltpu.SemaphoreType.DMA,
        ],
        compiler_params=pltpu.CompilerParams(use_tc_tiling_on_sc=False),
    )(x4, pe, table)
    return out


# speedup vs baseline: 1.0701x; 1.0053x over previous
import math

import jax
import jax.numpy as jnp
import numpy as np
from jax import lax
from jax.experimental import pallas as pl
from jax.experimental.pallas import tpu as pltpu
from jax.experimental.pallas import tpu_sc as plsc

VOCAB_SIZE = 1000000
DIM = 64
MAX_SEQ_LEN = 200
BATCH = 1024
SEQ_LEN = 200

_NC = 2
_NS = 16
_NW = _NC * _NS
_NBB = BATCH // 128
_NLG = _NW // _NBB
_LPG = SEQ_LEN // _NLG
_BCH = 128 // 16


def _position_encoding() -> np.ndarray:
    i = np.arange(MAX_SEQ_LEN, dtype=np.float64)[:, None]
    j = np.arange(DIM, dtype=np.float64)[None, :]
    even_mask = (np.arange(DIM) % 2 == 0)[None, :]
    temp_even = np.exp(-(j / DIM) * math.log(10000.0))
    temp_odd = np.exp(-((j - 1.0) / DIM) * math.log(10000.0))
    pe = np.where(even_mask, np.sin(i * temp_even), np.cos(i * temp_odd))
    return pe[:SEQ_LEN].astype(np.float32)


def _body(xt2_hbm, xtp_hbm, pes_hbm, t2_hbm, out_hbm,
          idx_v, par_v, pes_v, g0, g1, o0, o1, psem, gs0, gs1, ws0, ws1):
    c = lax.axis_index("c")
    s = lax.axis_index("s")
    wid = s * _NC + c
    bb = wid % _NBB
    lg = wid // _NBB
    col = bb * 128

    cp0 = pltpu.async_copy(xt2_hbm.at[lg, :, pl.ds(col, 128)], idx_v, psem)
    cp1 = pltpu.async_copy(xtp_hbm.at[lg, :, pl.ds(col, 128)], par_v, psem)
    cp2 = pltpu.async_copy(pes_hbm.at[lg], pes_v, psem)
    cp0.wait()
    cp1.wait()
    cp2.wait()

    def fire_gather(t, buf, sem):
        return pltpu.async_copy(t2_hbm.at[idx_v.at[t]], buf, sem)

    def wait_gather(buf, sem):
        pltpu.make_async_copy(t2_hbm.at[idx_v.at[0]], buf, sem).wait()

    def wait_wb(obuf, sem):
        pltpu.make_async_copy(obuf, out_hbm.at[0, :, pl.ds(col, 128)],
                              sem).wait()

    def compose(t, g, o):
        iota = lax.iota(jnp.int32, 16)
        rows = [iota + (16 * b) for b in range(_BCH)]
        pars = [par_v[t, pl.ds(16 * b, 16)] for b in range(_BCH)]

        @plsc.parallel_loop(0, DIM, unroll=2)
        def _(d):
            pv = pes_v[t, pl.ds(d * 16, 16)]
            for b in range(_BCH):
                vals = plsc.load_gather(g, [rows[b], pars[b] + d])
                o[d, pl.ds(16 * b, 16)] = vals + pv

    def fire_wb(t, obuf, sem):
        return pltpu.async_copy(
            obuf, out_hbm.at[lg * _LPG + t, :, pl.ds(col, 128)], sem)

    fire_gather(0, g0, gs0)
    fire_gather(1, g1, gs1)

    def step(i, carry):
        a = 2 * i
        bt = a + 1
        wait_gather(g0, gs0)

        @pl.when(i > 0)
        def _():
            wait_wb(o0, ws0)

        compose(a, g0, o0)

        @pl.when(i < _LPG // 2 - 1)
        def _():
            fire_gather(a + 2, g0, gs0)

        fire_wb(a, o0, ws0)

        wait_gather(g1, gs1)

        @pl.when(i > 0)
        def _():
            wait_wb(o1, ws1)

        compose(bt, g1, o1)

        @pl.when(i < _LPG // 2 - 1)
        def _():
            fire_gather(bt + 2, g1, gs1)

        fire_wb(bt, o1, ws1)
        return carry

    lax.fori_loop(0, _LPG // 2, step, 0)
    wait_wb(o0, ws0)
    wait_wb(o1, ws1)


def kernel(x, table):
    pe = _position_encoding()
    pes = np.broadcast_to(pe[:, :, None], (SEQ_LEN, DIM, 16))
    pes = jnp.asarray(
        pes.reshape(_NLG, _LPG, DIM * 16), dtype=jnp.float32)
    xt2 = ((x >> 1).T).reshape(_NLG, _LPG, BATCH)
    xtp = (((x & 1) << 6).T).reshape(_NLG, _LPG, BATCH)
    t2 = table.reshape(VOCAB_SIZE // 2, 128)
    mesh = plsc.VectorSubcoreMesh(core_axis_name="c", subcore_axis_name="s")
    out3 = pl.kernel(
        _body,
        out_type=jax.ShapeDtypeStruct((SEQ_LEN, DIM, BATCH), jnp.float32),
        mesh=mesh,
        scratch_types=[
            pltpu.VMEM((_LPG, 128), jnp.int32),
            pltpu.VMEM((_LPG, 128), jnp.int32),
            pltpu.VMEM((_LPG, DIM * 16), jnp.float32),
            pltpu.VMEM((128, 128), jnp.float32),
            pltpu.VMEM((128, 128), jnp.float32),
            pltpu.VMEM((DIM, 128), jnp.float32),
            pltpu.VMEM((DIM, 128), jnp.float32),
            pltpu.SemaphoreType.DMA,
            pltpu.SemaphoreType.DMA,
            pltpu.SemaphoreType.DMA,
            pltpu.SemaphoreType.DMA,
            pltpu.SemaphoreType.DMA,
        ],
        compiler_params=pltpu.CompilerParams(needs_layout_passes=False),
    )(xt2, xtp, pes, t2)
    return out3.transpose(2, 0, 1)
